# Initial kernel scaffold; baseline (speedup 1.0000x reference)
import jax
import jax.numpy as jnp
from jax.experimental import pallas as pl


def kernel(rois, fpn_class, fpn_bbox, image_meta):
    raise NotImplementedError("write your pallas kernel here")


# speedup vs baseline: 16.2531x; 16.2531x over previous
import functools

import jax
import jax.numpy as jnp
from jax.experimental import pallas as pl


_MAX_INST = 100
_MIN_CONF = 0.7
_NMS_THR = 0.3
_BBOX_STD4 = (0.1, 0.1, 0.2, 0.2)


def _refine_kernel(rois_ref, probs_ref, bbox_ref, win_ref, out_ref, *, n, c):
    probs = probs_ref[0]
    m = jnp.max(probs, axis=1, keepdims=True)
    ci = jax.lax.broadcasted_iota(jnp.int32, (n, c), 1)
    cid = jnp.min(jnp.where(probs == m, ci, c), axis=1, keepdims=True)

    bb = bbox_ref[0]
    ji = jax.lax.broadcasted_iota(jnp.int32, (n, 4 * c), 1)
    sel = (ji // 4) == cid
    masked = jnp.where(sel, bb, 0.0)
    k4 = ji % 4
    d = [jnp.sum(jnp.where(k4 == kk, masked, 0.0), axis=1, keepdims=True)
         for kk in range(4)]
    dy = d[0] * _BBOX_STD4[0]
    dx = d[1] * _BBOX_STD4[1]
    dh = d[2] * _BBOX_STD4[2]
    dw = d[3] * _BBOX_STD4[3]

    r = rois_ref[0]
    y1r, x1r, y2r, x2r = r[:, 0:1], r[:, 1:2], r[:, 2:3], r[:, 3:4]
    h = y2r - y1r
    w = x2r - x1r
    cy = y1r + 0.5 * h + dy * h
    cx = x1r + 0.5 * w + dx * w
    h = h * jnp.exp(dh)
    w = w * jnp.exp(dw)
    y1 = cy - 0.5 * h
    x1 = cx - 0.5 * w
    y2 = y1 + h
    x2 = x1 + w

    wy1 = win_ref[0, 0, 0]
    wx1 = win_ref[0, 0, 1]
    wy2 = win_ref[0, 0, 2]
    wx2 = win_ref[0, 0, 3]
    y1 = jnp.clip(y1, wy1, wy2)
    x1 = jnp.clip(x1, wx1, wx2)
    y2 = jnp.clip(y2, wy1, wy2)
    x2 = jnp.clip(x2, wx1, wx2)

    prek = (cid > 0) & (m >= _MIN_CONF)
    sc0 = jnp.where(prek, m, -1.0)
    cidf = cid.astype(jnp.float32)
    zero = jnp.zeros_like(m)
    out_ref[0] = jnp.concatenate(
        [y1, x1, y2, x2, sc0, cidf, zero, zero], axis=1)


def _nms_kernel(data_ref, out_ref, *, rows, lanes):
    y1 = data_ref[0, 0 * rows:1 * rows]
    x1 = data_ref[0, 1 * rows:2 * rows]
    y2 = data_ref[0, 2 * rows:3 * rows]
    x2 = data_ref[0, 3 * rows:4 * rows]
    sc_init = data_ref[0, 4 * rows:5 * rows]
    cidf = data_ref[0, 5 * rows:6 * rows]
    area = (y2 - y1) * (x2 - x1)
    gi = (jax.lax.broadcasted_iota(jnp.int32, (rows, lanes), 0) * lanes
          + jax.lax.broadcasted_iota(jnp.int32, (rows, lanes), 1))

    def body(i, carry):
        sc, det = carry
        m = jnp.max(sc)
        j = jnp.min(jnp.where(sc == m, gi, jnp.int32(1 << 30)))
        valid = m >= 0.0
        selm = gi == j

        def ext(a):
            return jnp.sum(jnp.where(selm, a, 0.0))

        by1 = ext(y1)
        bx1 = ext(x1)
        by2 = ext(y2)
        bx2 = ext(x2)
        bcid = ext(cidf)

        yy1 = jnp.maximum(by1, y1)
        xx1 = jnp.maximum(bx1, x1)
        yy2 = jnp.minimum(by2, y2)
        xx2 = jnp.minimum(bx2, x2)
        inter = jnp.maximum(yy2 - yy1, 0.0) * jnp.maximum(xx2 - xx1, 0.0)
        barea = (by2 - by1) * (bx2 - bx1)
        union = jnp.maximum(area + barea - inter, 1e-10)
        iou = inter / union
        supp = valid & (iou > _NMS_THR) & (cidf == bcid)
        sc = jnp.where(supp, -1.0, sc)

        vf = jnp.where(valid, 1.0, 0.0)
        row_i = jax.lax.broadcasted_iota(jnp.int32, (128, 8), 0)
        col_i = jax.lax.broadcasted_iota(jnp.int32, (128, 8), 1)
        newrow = jnp.where(
            col_i == 0, by1 * vf,
            jnp.where(col_i == 1, bx1 * vf,
                      jnp.where(col_i == 2, by2 * vf,
                                jnp.where(col_i == 3, bx2 * vf,
                                          jnp.where(col_i == 4, bcid * vf,
                                                    jnp.where(col_i == 5,
                                                              m * vf, 0.0))))))
        det = jnp.where(row_i == i, newrow, det)
        return sc, det

    _, det = jax.lax.fori_loop(
        0, _MAX_INST, body, (sc_init, jnp.zeros((128, 8), jnp.float32)))
    out_ref[0] = det


def kernel(rois, fpn_class, fpn_bbox, image_meta):
    b, n, c = fpn_class.shape

    image_shape = image_meta[0, 4:7]
    h, w = image_shape[0], image_shape[1]
    scale = jnp.stack([h - 1.0, w - 1.0, h - 1.0, w - 1.0])
    shift = jnp.array([0.0, 0.0, 1.0, 1.0], dtype=jnp.float32)
    window = ((image_meta[:, 7:11] - shift) / scale).reshape(b, 1, 4)

    bbox_flat = fpn_bbox.reshape(b, n, c * 4)

    packed = pl.pallas_call(
        functools.partial(_refine_kernel, n=n, c=c),
        grid=(b,),
        in_specs=[
            pl.BlockSpec((1, n, 4), lambda i: (i, 0, 0)),
            pl.BlockSpec((1, n, c), lambda i: (i, 0, 0)),
            pl.BlockSpec((1, n, c * 4), lambda i: (i, 0, 0)),
            pl.BlockSpec((1, 1, 4), lambda i: (i, 0, 0)),
        ],
        out_specs=pl.BlockSpec((1, n, 8), lambda i: (i, 0, 0)),
        out_shape=jax.ShapeDtypeStruct((b, n, 8), jnp.float32),
    )(rois, fpn_class, bbox_flat, window)

    lanes = 640
    n_pad = ((n + 8 * lanes - 1) // (8 * lanes)) * (8 * lanes)
    padded = jnp.pad(packed, ((0, 0), (0, n_pad - n), (0, 0)),
                     constant_values=-1.0)
    tiled = padded.transpose(0, 2, 1).reshape(b, 8 * (n_pad // lanes), lanes)
    rows = n_pad // lanes

    det = pl.pallas_call(
        functools.partial(_nms_kernel, rows=rows, lanes=lanes),
        grid=(b,),
        in_specs=[pl.BlockSpec((1, 8 * rows, lanes), lambda i: (i, 0, 0))],
        out_specs=pl.BlockSpec((1, 128, 8), lambda i: (i, 0, 0)),
        out_shape=jax.ShapeDtypeStruct((b, 128, 8), jnp.float32),
    )(tiled)

    return det[:, :_MAX_INST, :6]


# speedup vs baseline: 16.5715x; 1.0196x over previous
import functools

import jax
import jax.numpy as jnp
from jax.experimental import pallas as pl
from jax.experimental.pallas import tpu as pltpu


_MAX_INST = 100
_MIN_CONF = 0.7
_NMS_THR = 0.3


def _refine_kernel(rois_ref, probs_ref, bbox_ref, win_ref, out_ref, *, n, c):
    probs = probs_ref[0]
    m = jnp.max(probs, axis=1, keepdims=True)
    ci = jax.lax.broadcasted_iota(jnp.int32, (n, c), 1)
    cid = jnp.min(jnp.where(probs == m, ci, c), axis=1, keepdims=True)

    bb = bbox_ref[0]
    ji = jax.lax.broadcasted_iota(jnp.int32, (n, 4 * c), 1)
    masked = jnp.where((ji // 4) == cid, bb, 0.0)
    k4 = ji % 4
    d = [jnp.sum(jnp.where(k4 == kk, masked, 0.0), axis=1, keepdims=True)
         for kk in range(4)]
    d01 = jnp.concatenate([d[0], d[1]], axis=1) * 0.1
    d23 = jnp.concatenate([d[2], d[3]], axis=1) * 0.2

    r = rois_ref[0]
    p12 = r[:, 0:2]
    p34 = r[:, 2:4]
    hw = p34 - p12
    ctr = p12 + 0.5 * hw + d01 * hw
    hw2 = hw * jnp.exp(d23)
    tl = ctr - 0.5 * hw2
    br = tl + hw2

    wy1 = win_ref[0, 0, 0]
    wx1 = win_ref[0, 0, 1]
    wy2 = win_ref[0, 0, 2]
    wx2 = win_ref[0, 0, 3]
    li2 = jax.lax.broadcasted_iota(jnp.int32, (1, 2), 1)
    lo = jnp.where(li2 == 0, wy1, wx1)
    hi = jnp.where(li2 == 0, wy2, wx2)
    tl = jnp.clip(tl, lo, hi)
    br = jnp.clip(br, lo, hi)

    prek = (cid > 0) & (m >= _MIN_CONF)
    sc0 = jnp.where(prek, m, -1.0)
    cidf = cid.astype(jnp.float32)
    zero = jnp.zeros_like(m)
    out_ref[0] = jnp.concatenate([tl, br, sc0, cidf, zero, zero], axis=1)


def _nms_kernel(data_ref, rows_ref, out_ref, *, rows, lanes):
    y1 = data_ref[0, 0 * rows:1 * rows]
    x1 = data_ref[0, 1 * rows:2 * rows]
    y2 = data_ref[0, 2 * rows:3 * rows]
    x2 = data_ref[0, 3 * rows:4 * rows]
    sc_init = data_ref[0, 4 * rows:5 * rows]
    cidf = data_ref[0, 5 * rows:6 * rows]
    area = (y2 - y1) * (x2 - x1)
    gi = (jax.lax.broadcasted_iota(jnp.int32, (rows, lanes), 0) * lanes
          + jax.lax.broadcasted_iota(jnp.int32, (rows, lanes), 1))
    li8 = jax.lax.broadcasted_iota(jnp.int32, (1, 8), 1)
    ri = jax.lax.broadcasted_iota(jnp.int32, (8, 128), 0)
    li = jax.lax.broadcasted_iota(jnp.int32, (8, 128), 1)

    def body(i, carry):
        sc, det = carry
        m = jnp.max(sc)
        j = jnp.min(jnp.where(sc == m, gi, jnp.int32(1 << 30)))
        valid = m >= 0.0

        row = rows_ref[0, pl.ds(j, 1), :]

        def ext(kk):
            return jnp.sum(jnp.where(li8 == kk, row, 0.0))

        by1 = ext(0)
        bx1 = ext(1)
        by2 = ext(2)
        bx2 = ext(3)
        bcid = ext(5)

        yy1 = jnp.maximum(by1, y1)
        xx1 = jnp.maximum(bx1, x1)
        yy2 = jnp.minimum(by2, y2)
        xx2 = jnp.minimum(bx2, x2)
        inter = jnp.maximum(yy2 - yy1, 0.0) * jnp.maximum(xx2 - xx1, 0.0)
        barea = (by2 - by1) * (bx2 - bx1)
        union = jnp.maximum(area + barea - inter, 1e-10)
        iou = inter / union
        supp = valid & (iou > _NMS_THR) & (cidf == bcid)
        sc = jnp.where(supp, -1.0, sc)

        vf = jnp.where(valid, 1.0, 0.0)
        newcol = jnp.where(
            ri == 0, by1 * vf,
            jnp.where(ri == 1, bx1 * vf,
                      jnp.where(ri == 2, by2 * vf,
                                jnp.where(ri == 3, bx2 * vf,
                                          jnp.where(ri == 4, bcid * vf,
                                                    m * vf)))))
        det = jnp.where(li == i, newcol, det)
        return sc, det

    _, det = jax.lax.fori_loop(
        0, _MAX_INST, body, (sc_init, jnp.zeros((8, 128), jnp.float32)))
    out_ref[0] = det


def kernel(rois, fpn_class, fpn_bbox, image_meta):
    b, n, c = fpn_class.shape

    image_shape = image_meta[0, 4:7]
    h, w = image_shape[0], image_shape[1]
    scale = jnp.stack([h - 1.0, w - 1.0, h - 1.0, w - 1.0])
    shift = jnp.array([0.0, 0.0, 1.0, 1.0], dtype=jnp.float32)
    window = ((image_meta[:, 7:11] - shift) / scale).reshape(b, 1, 4)

    bbox_flat = fpn_bbox.reshape(b, n, c * 4)
    parallel = pltpu.CompilerParams(dimension_semantics=("parallel",))

    packed = pl.pallas_call(
        functools.partial(_refine_kernel, n=n, c=c),
        grid=(b,),
        in_specs=[
            pl.BlockSpec((1, n, 4), lambda i: (i, 0, 0)),
            pl.BlockSpec((1, n, c), lambda i: (i, 0, 0)),
            pl.BlockSpec((1, n, c * 4), lambda i: (i, 0, 0)),
            pl.BlockSpec((1, 1, 4), lambda i: (i, 0, 0)),
        ],
        out_specs=pl.BlockSpec((1, n, 8), lambda i: (i, 0, 0)),
        out_shape=jax.ShapeDtypeStruct((b, n, 8), jnp.float32),
        compiler_params=parallel,
    )(rois, fpn_class, bbox_flat, window)

    lanes = 640
    n_pad = ((n + 8 * lanes - 1) // (8 * lanes)) * (8 * lanes)
    padded = jnp.pad(packed, ((0, 0), (0, n_pad - n), (0, 0)),
                     constant_values=-1.0)
    tiled = padded.transpose(0, 2, 1).reshape(b, 8 * (n_pad // lanes), lanes)
    rows = n_pad // lanes

    det = pl.pallas_call(
        functools.partial(_nms_kernel, rows=rows, lanes=lanes),
        grid=(b,),
        in_specs=[
            pl.BlockSpec((1, 8 * rows, lanes), lambda i: (i, 0, 0)),
            pl.BlockSpec((1, n_pad, 8), lambda i: (i, 0, 0)),
        ],
        out_specs=pl.BlockSpec((1, 8, 128), lambda i: (i, 0, 0)),
        out_shape=jax.ShapeDtypeStruct((b, 8, 128), jnp.float32),
        compiler_params=parallel,
    )(tiled, padded)

    return det[:, :6, :_MAX_INST].transpose(0, 2, 1)


# speedup vs baseline: 17.9528x; 1.0833x over previous
import functools

import jax
import jax.numpy as jnp
from jax.experimental import pallas as pl
from jax.experimental.pallas import tpu as pltpu


_MAX_INST = 100
_MIN_CONF = 0.7
_NMS_THR = 0.3


def _refine_kernel(rois_ref, probs_ref, bbox_ref, win_ref, out_ref, *, n, c):
    probs = probs_ref[0]
    m = jnp.max(probs, axis=1, keepdims=True)
    ci = jax.lax.broadcasted_iota(jnp.int32, (n, c), 1)
    cid = jnp.min(jnp.where(probs == m, ci, c), axis=1, keepdims=True)

    bb = bbox_ref[0]
    ji = jax.lax.broadcasted_iota(jnp.int32, (n, 4 * c), 1)
    masked = jnp.where((ji // 4) == cid, bb, 0.0)
    k4 = ji % 4
    d = [jnp.sum(jnp.where(k4 == kk, masked, 0.0), axis=1, keepdims=True)
         for kk in range(4)]
    d01 = jnp.concatenate([d[0], d[1]], axis=1) * 0.1
    d23 = jnp.concatenate([d[2], d[3]], axis=1) * 0.2

    r = rois_ref[0]
    p12 = r[:, 0:2]
    p34 = r[:, 2:4]
    hw = p34 - p12
    ctr = p12 + 0.5 * hw + d01 * hw
    hw2 = hw * jnp.exp(d23)
    tl = ctr - 0.5 * hw2
    br = tl + hw2

    wy1 = win_ref[0, 0, 0]
    wx1 = win_ref[0, 0, 1]
    wy2 = win_ref[0, 0, 2]
    wx2 = win_ref[0, 0, 3]
    li2 = jax.lax.broadcasted_iota(jnp.int32, (1, 2), 1)
    lo = jnp.where(li2 == 0, wy1, wx1)
    hi = jnp.where(li2 == 0, wy2, wx2)
    tl = jnp.clip(tl, lo, hi)
    br = jnp.clip(br, lo, hi)

    prek = (cid > 0) & (m >= _MIN_CONF)
    sc0 = jnp.where(prek, m, -1.0)
    cidf = cid.astype(jnp.float32)
    zero = jnp.zeros_like(m)
    out_ref[0] = jnp.concatenate([tl, br, sc0, cidf, zero, zero], axis=1)


def _nms_kernel(data_ref, out_ref, *, b, rows, lanes):
    gi = (jax.lax.broadcasted_iota(jnp.int32, (rows, lanes), 0) * lanes
          + jax.lax.broadcasted_iota(jnp.int32, (rows, lanes), 1))
    ri = jax.lax.broadcasted_iota(jnp.int32, (8, 128), 0)
    li = jax.lax.broadcasted_iota(jnp.int32, (8, 128), 1)

    panels = []
    for bb_ in range(b):
        base = bb_ * 8 * rows
        y1 = data_ref[base + 0 * rows:base + 1 * rows, :]
        x1 = data_ref[base + 1 * rows:base + 2 * rows, :]
        y2 = data_ref[base + 2 * rows:base + 3 * rows, :]
        x2 = data_ref[base + 3 * rows:base + 4 * rows, :]
        sc0 = data_ref[base + 4 * rows:base + 5 * rows, :]
        cidf = data_ref[base + 5 * rows:base + 6 * rows, :]
        area = (y2 - y1) * (x2 - x1)
        panels.append((y1, x1, y2, x2, cidf, area, sc0))

    def one(i, y1, x1, y2, x2, cidf, area, sc, det):
        m = jnp.max(sc, axis=(0, 1), keepdims=True)
        selm = sc == m
        jsel = jnp.min(jnp.where(selm, gi, jnp.int32(1 << 30)),
                       axis=(0, 1), keepdims=True)
        sel1 = selm & (gi == jsel)
        valid = m >= 0.0

        def ext(a):
            return jnp.sum(jnp.where(sel1, a, 0.0), axis=(0, 1),
                           keepdims=True)

        by1 = ext(y1)
        bx1 = ext(x1)
        by2 = ext(y2)
        bx2 = ext(x2)
        bcid = ext(cidf)

        yy1 = jnp.maximum(by1, y1)
        xx1 = jnp.maximum(bx1, x1)
        yy2 = jnp.minimum(by2, y2)
        xx2 = jnp.minimum(bx2, x2)
        inter = jnp.maximum(yy2 - yy1, 0.0) * jnp.maximum(xx2 - xx1, 0.0)
        barea = (by2 - by1) * (bx2 - bx1)
        union = jnp.maximum(area + barea - inter, 1e-10)
        iou = inter / union
        supp = valid & (iou > _NMS_THR) & (cidf == bcid)
        sc = jnp.where(supp, -1.0, sc)

        vf = jnp.where(valid, 1.0, 0.0)
        newcol = jnp.where(
            ri == 0, by1 * vf,
            jnp.where(ri == 1, bx1 * vf,
                      jnp.where(ri == 2, by2 * vf,
                                jnp.where(ri == 3, bx2 * vf,
                                          jnp.where(ri == 4, bcid * vf,
                                                    m * vf)))))
        det = jnp.where(li == i, newcol, det)
        return sc, det

    def body(i, carry):
        out = []
        for bb_, (sc, det) in enumerate(carry):
            y1, x1, y2, x2, cidf, area, sc0 = panels[bb_]
            out.append(one(i, y1, x1, y2, x2, cidf, area, sc, det))
        return tuple(out)

    init = tuple((panels[bb_][6], jnp.zeros((8, 128), jnp.float32))
                 for bb_ in range(b))
    final = jax.lax.fori_loop(0, _MAX_INST, body, init)
    for bb_ in range(b):
        out_ref[bb_] = final[bb_][1]


def kernel(rois, fpn_class, fpn_bbox, image_meta):
    b, n, c = fpn_class.shape

    image_shape = image_meta[0, 4:7]
    h, w = image_shape[0], image_shape[1]
    scale = jnp.stack([h - 1.0, w - 1.0, h - 1.0, w - 1.0])
    shift = jnp.array([0.0, 0.0, 1.0, 1.0], dtype=jnp.float32)
    window = ((image_meta[:, 7:11] - shift) / scale).reshape(b, 1, 4)

    bbox_flat = fpn_bbox.reshape(b, n, c * 4)
    parallel = pltpu.CompilerParams(dimension_semantics=("parallel",))

    packed = pl.pallas_call(
        functools.partial(_refine_kernel, n=n, c=c),
        grid=(b,),
        in_specs=[
            pl.BlockSpec((1, n, 4), lambda i: (i, 0, 0)),
            pl.BlockSpec((1, n, c), lambda i: (i, 0, 0)),
            pl.BlockSpec((1, n, c * 4), lambda i: (i, 0, 0)),
            pl.BlockSpec((1, 1, 4), lambda i: (i, 0, 0)),
        ],
        out_specs=pl.BlockSpec((1, n, 8), lambda i: (i, 0, 0)),
        out_shape=jax.ShapeDtypeStruct((b, n, 8), jnp.float32),
        compiler_params=parallel,
    )(rois, fpn_class, bbox_flat, window)

    lanes = 640
    n_pad = ((n + 8 * lanes - 1) // (8 * lanes)) * (8 * lanes)
    padded = jnp.pad(packed, ((0, 0), (0, n_pad - n), (0, 0)),
                     constant_values=-1.0)
    tiled = padded.transpose(0, 2, 1).reshape(b * 8 * (n_pad // lanes), lanes)
    rows = n_pad // lanes

    det = pl.pallas_call(
        functools.partial(_nms_kernel, b=b, rows=rows, lanes=lanes),
        out_shape=jax.ShapeDtypeStruct((b, 8, 128), jnp.float32),
    )(tiled)

    return det[:, :6, :_MAX_INST].transpose(0, 2, 1)


# speedup vs baseline: 18.6621x; 1.0395x over previous
import functools

import jax
import jax.numpy as jnp
from jax.experimental import pallas as pl
from jax.experimental.pallas import tpu as pltpu


_MAX_INST = 100
_MIN_CONF = 0.7
_NMS_THR = 0.3


def _refine_kernel(rois_ref, probs_ref, bbox_ref, win_ref, out_ref, *, n, c):
    probs = probs_ref[0]
    m = jnp.max(probs, axis=1, keepdims=True)
    ci = jax.lax.broadcasted_iota(jnp.int32, (n, c), 1)
    cid = jnp.min(jnp.where(probs == m, ci, c), axis=1, keepdims=True)

    bb = bbox_ref[0]
    ji = jax.lax.broadcasted_iota(jnp.int32, (n, 4 * c), 1)
    masked = jnp.where((ji // 4) == cid, bb, 0.0)
    srow = jax.lax.broadcasted_iota(jnp.int32, (4 * c, 4), 0)
    scol = jax.lax.broadcasted_iota(jnp.int32, (4 * c, 4), 1)
    smat = (srow % 4 == scol).astype(jnp.float32)
    deltas = jax.lax.dot_general(masked, smat, (((1,), (0,)), ((), ())),
                                 preferred_element_type=jnp.float32)
    d01 = deltas[:, 0:2] * 0.1
    d23 = deltas[:, 2:4] * 0.2

    r = rois_ref[0]
    p12 = r[:, 0:2]
    p34 = r[:, 2:4]
    hw = p34 - p12
    ctr = p12 + 0.5 * hw + d01 * hw
    hw2 = hw * jnp.exp(d23)
    tl = ctr - 0.5 * hw2
    br = tl + hw2

    wy1 = win_ref[0, 0, 0]
    wx1 = win_ref[0, 0, 1]
    wy2 = win_ref[0, 0, 2]
    wx2 = win_ref[0, 0, 3]
    li2 = jax.lax.broadcasted_iota(jnp.int32, (1, 2), 1)
    lo = jnp.where(li2 == 0, wy1, wx1)
    hi = jnp.where(li2 == 0, wy2, wx2)
    tl = jnp.clip(tl, lo, hi)
    br = jnp.clip(br, lo, hi)

    prek = (cid > 0) & (m >= _MIN_CONF)
    sc0 = jnp.where(prek, m, -1.0)
    cidf = cid.astype(jnp.float32)
    zero = jnp.zeros_like(m)
    out_ref[0] = jnp.concatenate([tl, br, sc0, cidf, zero, zero], axis=1)


def _nms_kernel(data_ref, out_ref, *, b, rows, lanes):
    gi = (jax.lax.broadcasted_iota(jnp.int32, (rows, lanes), 0) * lanes
          + jax.lax.broadcasted_iota(jnp.int32, (rows, lanes), 1))
    ri = jax.lax.broadcasted_iota(jnp.int32, (8, 128), 0)
    li = jax.lax.broadcasted_iota(jnp.int32, (8, 128), 1)

    panels = []
    for bb_ in range(b):
        base = bb_ * 8 * rows
        y1 = data_ref[base + 0 * rows:base + 1 * rows, :]
        x1 = data_ref[base + 1 * rows:base + 2 * rows, :]
        y2 = data_ref[base + 2 * rows:base + 3 * rows, :]
        x2 = data_ref[base + 3 * rows:base + 4 * rows, :]
        sc0 = data_ref[base + 4 * rows:base + 5 * rows, :]
        cidf = data_ref[base + 5 * rows:base + 6 * rows, :]
        area = (y2 - y1) * (x2 - x1)
        panels.append((y1, x1, y2, x2, cidf, area, sc0))

    def one(i, y1, x1, y2, x2, cidf, area, sc, det):
        m = jnp.max(sc, axis=(0, 1), keepdims=True)
        selm = sc == m
        jsel = jnp.min(jnp.where(selm, gi, jnp.int32(1 << 30)),
                       axis=(0, 1), keepdims=True)
        sel1 = selm & (gi == jsel)
        valid = m >= 0.0

        def ext(a):
            return jnp.sum(jnp.where(sel1, a, 0.0), axis=(0, 1),
                           keepdims=True)

        by1 = ext(y1)
        bx1 = ext(x1)
        by2 = ext(y2)
        bx2 = ext(x2)
        bcid = ext(cidf)

        yy1 = jnp.maximum(by1, y1)
        xx1 = jnp.maximum(bx1, x1)
        yy2 = jnp.minimum(by2, y2)
        xx2 = jnp.minimum(bx2, x2)
        inter = jnp.maximum(yy2 - yy1, 0.0) * jnp.maximum(xx2 - xx1, 0.0)
        barea = (by2 - by1) * (bx2 - bx1)
        union = jnp.maximum(area + barea - inter, 1e-10)
        iou = inter / union
        supp = valid & (iou > _NMS_THR) & (cidf == bcid)
        sc = jnp.where(supp, -1.0, sc)

        vf = jnp.where(valid, 1.0, 0.0)
        newcol = jnp.where(
            ri == 0, by1 * vf,
            jnp.where(ri == 1, bx1 * vf,
                      jnp.where(ri == 2, by2 * vf,
                                jnp.where(ri == 3, bx2 * vf,
                                          jnp.where(ri == 4, bcid * vf,
                                                    m * vf)))))
        det = jnp.where(li == i, newcol, det)
        return sc, det

    def body(i, carry):
        out = []
        for bb_, (sc, det) in enumerate(carry):
            y1, x1, y2, x2, cidf, area, sc0 = panels[bb_]
            out.append(one(i, y1, x1, y2, x2, cidf, area, sc, det))
        return tuple(out)

    init = tuple((panels[bb_][6], jnp.zeros((8, 128), jnp.float32))
                 for bb_ in range(b))
    final = jax.lax.fori_loop(0, _MAX_INST, body, init)
    for bb_ in range(b):
        out_ref[bb_] = final[bb_][1]


def kernel(rois, fpn_class, fpn_bbox, image_meta):
    b, n, c = fpn_class.shape

    image_shape = image_meta[0, 4:7]
    h, w = image_shape[0], image_shape[1]
    scale = jnp.stack([h - 1.0, w - 1.0, h - 1.0, w - 1.0])
    shift = jnp.array([0.0, 0.0, 1.0, 1.0], dtype=jnp.float32)
    window = ((image_meta[:, 7:11] - shift) / scale).reshape(b, 1, 4)

    bbox_flat = fpn_bbox.reshape(b, n, c * 4)
    parallel = pltpu.CompilerParams(dimension_semantics=("parallel",))

    packed = pl.pallas_call(
        functools.partial(_refine_kernel, n=n, c=c),
        grid=(b,),
        in_specs=[
            pl.BlockSpec((1, n, 4), lambda i: (i, 0, 0)),
            pl.BlockSpec((1, n, c), lambda i: (i, 0, 0)),
            pl.BlockSpec((1, n, c * 4), lambda i: (i, 0, 0)),
            pl.BlockSpec((1, 1, 4), lambda i: (i, 0, 0)),
        ],
        out_specs=pl.BlockSpec((1, n, 8), lambda i: (i, 0, 0)),
        out_shape=jax.ShapeDtypeStruct((b, n, 8), jnp.float32),
        compiler_params=parallel,
    )(rois, fpn_class, bbox_flat, window)

    lanes = 640
    n_pad = ((n + 8 * lanes - 1) // (8 * lanes)) * (8 * lanes)
    padded = jnp.pad(packed, ((0, 0), (0, n_pad - n), (0, 0)),
                     constant_values=-1.0)
    tiled = padded.transpose(0, 2, 1).reshape(b * 8 * (n_pad // lanes), lanes)
    rows = n_pad // lanes

    det = pl.pallas_call(
        functools.partial(_nms_kernel, b=b, rows=rows, lanes=lanes),
        out_shape=jax.ShapeDtypeStruct((b, 8, 128), jnp.float32),
    )(tiled)

    return det[:, :6, :_MAX_INST].transpose(0, 2, 1)
